# baseline (speedup 1.0000x reference)
import functools

import jax
import jax.numpy as jnp
from jax import lax
from jax.experimental import pallas as pl
from jax.experimental.pallas import tpu as pltpu
from jax.experimental.pallas import tpu_sc as plsc

V = 100000
D = 300
DP = 304
B = 1024
NFULL = 18
TAIL_OFF = 284


def _accumulate_rows(buf, nrows, tail_mask):
    def body(r, accs):
        new = []
        for i in range(NFULL):
            new.append(accs[i] + buf[r, pl.ds(i * 16, 16)])
        v = buf[r, pl.ds(TAIL_OFF, 16)]
        new.append(accs[NFULL] + jnp.where(tail_mask, v, 0.0))
        return tuple(new)
    init = tuple(jnp.zeros((16,), jnp.float32) for _ in range(NFULL + 1))
    return lax.fori_loop(0, nrows, body, init)


def _make_sc_pool():
    info = plsc.get_sparse_core_info()
    nc, ns = info.num_cores, info.num_subcores
    nw = nc * ns
    bw = B // nw

    mesh = plsc.VectorSubcoreMesh(core_axis_name="c", subcore_axis_name="s")

    @functools.partial(
        pl.kernel,
        mesh=mesh,
        compiler_params=pltpu.CompilerParams(use_tc_tiling_on_sc=False),
        out_type=jax.ShapeDtypeStruct((4, B, DP), jnp.float32),
        scratch_types=[
            pltpu.VMEM((bw, 24), jnp.int32),
            pltpu.VMEM((bw, 200), jnp.int32),
            pltpu.VMEM((bw, 24), jnp.int32),
            pltpu.VMEM((bw, 200), jnp.int32),
            pltpu.VMEM((200, D), jnp.float32),
            pltpu.VMEM((DP,), jnp.float32),
            pltpu.SemaphoreType.DMA,
        ],
    )
    def sc_pool(jt_idx, jd_idx, ct_idx, cr_idx,
                t_jt, t_jd, t_ct, t_cr, out,
                iv_jt, iv_jd, iv_ct, iv_cr, rows, accv, sem):
        wid = lax.axis_index("s") * nc + lax.axis_index("c")
        base = wid * bw

        pltpu.sync_copy(jt_idx.at[pl.ds(base, bw)], iv_jt)
        pltpu.sync_copy(jd_idx.at[pl.ds(base, bw)], iv_jd)
        pltpu.sync_copy(ct_idx.at[pl.ds(base, bw)], iv_ct)
        pltpu.sync_copy(cr_idx.at[pl.ds(base, bw)], iv_cr)

        tail_mask = lax.iota(jnp.int32, 16) >= 4

        def col_body(j, carry):
            for t, (iv, tbl, n, npad) in enumerate((
                    (iv_jt, t_jt, 20, 24),
                    (iv_jd, t_jd, 200, 200),
                    (iv_ct, t_ct, 20, 24),
                    (iv_cr, t_cr, 200, 200))):
                pltpu.async_copy(
                    tbl.at[iv.at[j]], rows.at[pl.ds(0, npad)], sem).wait()
                accs = _accumulate_rows(rows, n, tail_mask)
                accv[pl.ds(288, 16)] = jnp.zeros((16,), jnp.float32)
                accv[pl.ds(TAIL_OFF, 16)] = accs[NFULL]
                accv[pl.ds(272, 16)] = accs[17]
                for i in range(17):
                    accv[pl.ds(i * 16, 16)] = accs[i]
                pltpu.sync_copy(accv, out.at[t, base + j])
            return carry

        lax.fori_loop(0, bw, col_body, 0)

    return sc_pool


def _mlp_body(jt, jd, ct, cr, pooled, w1p, b1, w2, b2, w3, b3, out):
    h = jnp.broadcast_to(b1[...], (B, 400))
    for t, idx in enumerate((jt, jd, ct, cr)):
        cnt = jnp.sum((idx[...] != 1).astype(jnp.float32), axis=0)
        x = pooled[t] / cnt[:, None]
        h = h + jnp.dot(x, w1p[t], preferred_element_type=jnp.float32)
    h = jax.nn.relu(h)
    h = jax.nn.relu(jnp.dot(h, w2[...], preferred_element_type=jnp.float32)
                    + b2[...])
    h = jax.nn.relu(jnp.dot(h, w3[...], preferred_element_type=jnp.float32)
                    + b3[...])
    out[...] = jax.nn.sigmoid(h)


def kernel(job_title, job_description, candidate_title, candidate_resume,
           emb_job_title, emb_job_description, emb_candidate_title,
           emb_candidate_resume, W1, b1, W2, b2, W3, b3):
    jt = job_title.astype(jnp.int32)
    jd = job_description.astype(jnp.int32)
    ct = candidate_title.astype(jnp.int32)
    cr = candidate_resume.astype(jnp.int32)

    jt_t = jnp.pad(jt.T, ((0, 0), (0, 4)))
    jd_t = jd.T
    ct_t = jnp.pad(ct.T, ((0, 0), (0, 4)))
    cr_t = cr.T

    sc_pool = _make_sc_pool()
    pooled = sc_pool(jt_t, jd_t, ct_t, cr_t,
                     emb_job_title, emb_job_description,
                     emb_candidate_title, emb_candidate_resume)

    w1p = jnp.zeros((4, DP, 400), jnp.float32)
    w1p = w1p.at[:, :D, :].set(W1.reshape(4, D, 400))

    out = pl.pallas_call(
        _mlp_body,
        out_shape=jax.ShapeDtypeStruct((B, 1), jnp.float32),
    )(jt, jd, ct, cr, pooled, w1p,
      b1.reshape(1, 400), W2, b2.reshape(1, 100), W3, b3.reshape(1, 1))
    return out


# speedup vs baseline: 1.0269x; 1.0269x over previous
import functools

import jax
import jax.numpy as jnp
from jax import lax
from jax.experimental import pallas as pl
from jax.experimental.pallas import tpu as pltpu
from jax.experimental.pallas import tpu_sc as plsc

V = 100000
D = 300
DP = 304
B = 1024
NFULL = 18
TAIL_OFF = 284
CHUNK = 104


def _accum_rows(buf, nrows, tail_mask, accs):
    def body(r, a):
        new = []
        for i in range(NFULL):
            new.append(a[i] + buf[r, pl.ds(i * 16, 16)])
        v = buf[r, pl.ds(TAIL_OFF, 16)]
        new.append(a[NFULL] + jnp.where(tail_mask, v, 0.0))
        return tuple(new)
    return lax.fori_loop(0, nrows, body, accs)


def _zero_accs():
    return tuple(jnp.zeros((16,), jnp.float32) for _ in range(NFULL + 1))


def _make_sc_pool():
    info = plsc.get_sparse_core_info()
    nc, ns = info.num_cores, info.num_subcores
    nw = nc * ns
    bw = B // nw

    mesh = plsc.VectorSubcoreMesh(core_axis_name="c", subcore_axis_name="s")

    @functools.partial(
        pl.kernel,
        mesh=mesh,
        compiler_params=pltpu.CompilerParams(use_tc_tiling_on_sc=False,
                                             needs_layout_passes=False),
        out_type=jax.ShapeDtypeStruct((4, B, DP), jnp.float32),
        scratch_types=[
            pltpu.VMEM((200, 32), jnp.int32),
            pltpu.VMEM((32, 24), jnp.int32),
            pltpu.VMEM((32, 200), jnp.int32),
            pltpu.VMEM((32, 24), jnp.int32),
            pltpu.VMEM((32, 200), jnp.int32),
            pltpu.VMEM((CHUNK, D), jnp.float32),
            pltpu.VMEM((CHUNK, D), jnp.float32),
            pltpu.VMEM((DP,), jnp.float32),
            pltpu.VMEM((DP,), jnp.float32),
            pltpu.SemaphoreType.DMA,
            pltpu.SemaphoreType.DMA,
            pltpu.SemaphoreType.DMA,
            pltpu.SemaphoreType.DMA,
        ],
    )
    def sc_pool(jt_idx, jd_idx, ct_idx, cr_idx,
                t_jt, t_jd, t_ct, t_cr, out,
                iv2d, iv_jt, iv_jd, iv_ct, iv_cr,
                buf_a, buf_b, acc_a, acc_b,
                sem_a, sem_b, sem_oa, sem_ob):
        wid = lax.axis_index("s") * nc + lax.axis_index("c")
        base = wid * bw

        lanes = lax.iota(jnp.int32, 16)
        tail_mask = lanes >= 4

        for idx_hbm, iv_t, n, npad in (
                (jt_idx, iv_jt, 20, 24), (jd_idx, iv_jd, 200, 200),
                (ct_idx, iv_ct, 20, 24), (cr_idx, iv_cr, 200, 200)):
            pltpu.sync_copy(idx_hbm.at[:, pl.ds(base, bw)],
                            iv2d.at[pl.ds(0, n)])

            def tr_body(r, _, iv_t=iv_t, n=n):
                col = jnp.full((16,), 0, jnp.int32) + r
                v0 = jnp.where(r < n, iv2d[r, pl.ds(0, 16)], 0)
                v1 = jnp.where(r < n, iv2d[r, pl.ds(16, 16)], 0)
                plsc.store_scatter(iv_t, [lanes, col], v0)
                plsc.store_scatter(iv_t, [lanes + 16, col], v1)
                return 0
            lax.fori_loop(0, npad, tr_body, 0)

        stages = ((0, iv_jt, t_jt, 0, 24, 20, True),
                  (1, iv_jd, t_jd, 0, 104, 104, False),
                  (1, iv_jd, t_jd, 104, 96, 96, True),
                  (2, iv_ct, t_ct, 0, 24, 20, True),
                  (3, iv_cr, t_cr, 0, 104, 104, False),
                  (3, iv_cr, t_cr, 104, 96, 96, True))
        bufs = (buf_a, buf_b)
        sems = (sem_a, sem_b)
        accvs = (acc_a, acc_b)
        osems = (sem_oa, sem_ob)

        def gather_cp(s, col, b):
            _, iv, tbl, off, ng, _, _ = stages[s]
            return pltpu.make_async_copy(
                tbl.at[iv.at[col, pl.ds(off, ng)]],
                bufs[b].at[pl.ds(0, ng)], sems[b])

        def out_cp(t, col, p):
            return pltpu.make_async_copy(
                accvs[p], out.at[t, base + col], osems[p])

        out_cp(0, 0, 0).start()
        out_cp(1, 0, 1).start()
        gather_cp(0, 0, 0).start()

        def col_body(j, carry):
            accs = _zero_accs()
            fin = 0
            for s, (t, iv, tbl, off, ng, n, last) in enumerate(stages):
                ns = (s + 1) % len(stages)
                ncol = j if s + 1 < len(stages) else jnp.minimum(j + 1, bw - 1)
                gather_cp(ns, ncol, (s + 1) % 2).start()
                gather_cp(s, j, s % 2).wait()
                accs = _accum_rows(bufs[s % 2], n, tail_mask, accs)
                if last:
                    p = fin % 2
                    accv = accvs[p]
                    out_cp(t, j, p).wait()
                    accv[pl.ds(288, 16)] = jnp.zeros((16,), jnp.float32)
                    accv[pl.ds(TAIL_OFF, 16)] = accs[NFULL]
                    accv[pl.ds(272, 16)] = accs[17]
                    for i in range(17):
                        accv[pl.ds(i * 16, 16)] = accs[i]
                    out_cp(t, j, p).start()
                    accs = _zero_accs()
                    fin += 1
            return carry

        lax.fori_loop(0, bw, col_body, 0)

        gather_cp(0, bw - 1, 0).wait()
        out_cp(0, bw - 1, 0).wait()
        out_cp(1, bw - 1, 1).wait()

    return sc_pool


def _mlp_body(jt, jd, ct, cr, pooled, w1, b1, w2, b2, w3, b3, out):
    h = jnp.broadcast_to(b1[...], (B, 400))
    zpad = jnp.zeros((DP - D, 400), jnp.float32)
    for t, idx in enumerate((jt, jd, ct, cr)):
        cnt = jnp.sum((idx[...] != 1).astype(jnp.float32), axis=0)
        x = pooled[t] / cnt[:, None]
        w1t = jnp.concatenate([w1[pl.ds(t * D, D), :], zpad], axis=0)
        h = h + jnp.dot(x, w1t, preferred_element_type=jnp.float32)
    h = jax.nn.relu(h)
    h = jax.nn.relu(jnp.dot(h, w2[...], preferred_element_type=jnp.float32)
                    + b2[...])
    h = jax.nn.relu(jnp.dot(h, w3[...], preferred_element_type=jnp.float32)
                    + b3[...])
    out[...] = jax.nn.sigmoid(h)


def kernel(job_title, job_description, candidate_title, candidate_resume,
           emb_job_title, emb_job_description, emb_candidate_title,
           emb_candidate_resume, W1, b1, W2, b2, W3, b3):
    jt = job_title.astype(jnp.int32)
    jd = job_description.astype(jnp.int32)
    ct = candidate_title.astype(jnp.int32)
    cr = candidate_resume.astype(jnp.int32)

    sc_pool = _make_sc_pool()
    pooled = sc_pool(jt, jd, ct, cr,
                     emb_job_title, emb_job_description,
                     emb_candidate_title, emb_candidate_resume)

    out = pl.pallas_call(
        _mlp_body,
        out_shape=jax.ShapeDtypeStruct((B, 1), jnp.float32),
    )(jt, jd, ct, cr, pooled, W1,
      b1.reshape(1, 400), W2, b2.reshape(1, 100), W3, b3.reshape(1, 1))
    return out


# speedup vs baseline: 1.9827x; 1.9307x over previous
import functools

import jax
import jax.numpy as jnp
from jax import lax
from jax.experimental import pallas as pl
from jax.experimental.pallas import tpu as pltpu
from jax.experimental.pallas import tpu_sc as plsc

V = 100000
D = 300
DP = 384
B = 1024
NSL = 19
TPAD = 24
ND = 200
C0 = 104
C1 = 96
RB = 1000


def _pad_body(t_ref, o_ref):
    blk = t_ref[...]
    o_ref[...] = jnp.concatenate(
        [blk, jnp.zeros((RB, DP - D), jnp.float32)], axis=1)


def _pad_table(tbl):
    return pl.pallas_call(
        _pad_body,
        grid=(V // RB,),
        in_specs=[pl.BlockSpec((RB, D), lambda i: (i, 0))],
        out_specs=pl.BlockSpec((RB, DP), lambda i: (i, 0)),
        out_shape=jax.ShapeDtypeStruct((V, DP), jnp.float32),
    )(tbl)


def _accum_rows(buf, nrows, accs):
    def body(r, a):
        return tuple(a[i] + buf[r, pl.ds(i * 16, 16)] for i in range(NSL))
    return lax.fori_loop(0, nrows, body, accs)


def _zero_accs():
    return tuple(jnp.zeros((16,), jnp.float32) for _ in range(NSL))


def _make_sc_pool():
    info = plsc.get_sparse_core_info()
    nc, ns = info.num_cores, info.num_subcores
    nw = nc * ns
    bw = B // nw
    ng = bw // 8

    mesh = plsc.VectorSubcoreMesh(core_axis_name="c", subcore_axis_name="s")

    @functools.partial(
        pl.kernel,
        mesh=mesh,
        compiler_params=pltpu.CompilerParams(use_tc_tiling_on_sc=True,
                                             needs_layout_passes=False),
        out_type=jax.ShapeDtypeStruct((4, B, DP), jnp.float32),
        scratch_types=[
            pltpu.VMEM((B // nw * TPAD,), jnp.int32),
            pltpu.VMEM((B // nw * ND,), jnp.int32),
            pltpu.VMEM((B // nw * TPAD,), jnp.int32),
            pltpu.VMEM((B // nw * ND,), jnp.int32),
            pltpu.VMEM((C0, DP), jnp.float32),
            pltpu.VMEM((C0, DP), jnp.float32),
            pltpu.VMEM((4, 8, DP), jnp.float32),
            pltpu.SemaphoreType.DMA,
            pltpu.SemaphoreType.DMA,
            pltpu.SemaphoreType.DMA,
            pltpu.SemaphoreType.DMA,
            pltpu.SemaphoreType.DMA,
            pltpu.SemaphoreType.DMA,
        ],
    )
    def sc_pool(jt_idx, jd_idx, ct_idx, cr_idx,
                t_jt, t_jd, t_ct, t_cr, out,
                iv_jt, iv_jd, iv_ct, iv_cr,
                buf_a, buf_b, ostage,
                sem_a, sem_b, sem_f0, sem_f1, sem_f2, sem_f3):
        wid = lax.axis_index("s") * nc + lax.axis_index("c")
        base = wid * bw

        pltpu.sync_copy(jt_idx.at[pl.ds(base * TPAD, bw * TPAD)], iv_jt)
        pltpu.sync_copy(jd_idx.at[pl.ds(base * ND, bw * ND)], iv_jd)
        pltpu.sync_copy(ct_idx.at[pl.ds(base * TPAD, bw * TPAD)], iv_ct)
        pltpu.sync_copy(cr_idx.at[pl.ds(base * ND, bw * ND)], iv_cr)

        for t in range(4):
            for jm in range(8):
                for k in range(5):
                    ostage[t, jm, pl.ds(304 + k * 16, 16)] = (
                        jnp.zeros((16,), jnp.float32))

        stages = ((0, iv_jt, t_jt, TPAD, 0, TPAD, 20, True),
                  (1, iv_jd, t_jd, ND, 0, C0, C0, False),
                  (1, iv_jd, t_jd, ND, C0, C1, C1, True),
                  (2, iv_ct, t_ct, TPAD, 0, TPAD, 20, True),
                  (3, iv_cr, t_cr, ND, 0, C0, C0, False),
                  (3, iv_cr, t_cr, ND, C0, C1, C1, True))
        bufs = (buf_a, buf_b)
        sems = (sem_a, sem_b)
        fsems = (sem_f0, sem_f1, sem_f2, sem_f3)

        def gather_cp(s, col, b):
            _, iv, tbl, stride, off, n, _, _ = stages[s]
            return pltpu.make_async_copy(
                tbl.at[iv.at[pl.ds(col * stride + off, n)]],
                bufs[b].at[pl.ds(0, n)], sems[b])

        def flush_cp(t, g):
            return pltpu.make_async_copy(
                ostage.at[t], out.at[t, pl.ds(base + g * 8, 8)], fsems[t])

        for t in range(4):
            flush_cp(t, 0).start()
        gather_cp(0, 0, 0).start()

        def grp_body(g, carry):
            for t in range(4):
                flush_cp(t, g).wait()

            def col_body(jm, carry2):
                j = g * 8 + jm
                accs = _zero_accs()
                for s, (t, iv, tbl, stride, off, n, na, last) in (
                        enumerate(stages)):
                    nxt = (s + 1) % len(stages)
                    ncol = j if s + 1 < len(stages) else (
                        jnp.minimum(j + 1, bw - 1))
                    gather_cp(nxt, ncol, (s + 1) % 2).start()
                    gather_cp(s, j, s % 2).wait()
                    accs = _accum_rows(bufs[s % 2], na, accs)
                    if last:
                        for i in range(NSL):
                            ostage[t, jm, pl.ds(i * 16, 16)] = accs[i]
                        accs = _zero_accs()
                return carry2

            lax.fori_loop(0, 8, col_body, 0)
            for t in range(4):
                flush_cp(t, g).start()
            return carry

        lax.fori_loop(0, ng, grp_body, 0)

        gather_cp(0, bw - 1, 0).wait()
        for t in range(4):
            flush_cp(t, ng - 1).wait()

    return sc_pool


def _mlp_body(jt, jd, ct, cr, pooled, w1, b1, w2, b2, w3, b3, out):
    h = jnp.broadcast_to(b1[...], (B, 400))
    zpad = jnp.zeros((DP - D, 400), jnp.float32)
    for t, idx in enumerate((jt, jd, ct, cr)):
        cnt = jnp.sum((idx[...] != 1).astype(jnp.float32), axis=0)
        x = pooled[t] / cnt[:, None]
        w1t = jnp.concatenate([w1[pl.ds(t * D, D), :], zpad], axis=0)
        h = h + jnp.dot(x, w1t, preferred_element_type=jnp.float32)
    h = jax.nn.relu(h)
    h = jax.nn.relu(jnp.dot(h, w2[...], preferred_element_type=jnp.float32)
                    + b2[...])
    h = jax.nn.relu(jnp.dot(h, w3[...], preferred_element_type=jnp.float32)
                    + b3[...])
    out[...] = jax.nn.sigmoid(h)


def _flatten_idx(idx, npad):
    n = idx.shape[0]
    cols = idx.T
    if npad > n:
        k = jnp.arange(npad - n, dtype=jnp.int32)[None, :]
        c = jnp.arange(B, dtype=jnp.int32)[:, None]
        fill = (c * 4 + k) % V
        cols = jnp.concatenate([cols, fill], axis=1)
    return cols.reshape(-1)


def kernel(job_title, job_description, candidate_title, candidate_resume,
           emb_job_title, emb_job_description, emb_candidate_title,
           emb_candidate_resume, W1, b1, W2, b2, W3, b3):
    jt = job_title.astype(jnp.int32)
    jd = job_description.astype(jnp.int32)
    ct = candidate_title.astype(jnp.int32)
    cr = candidate_resume.astype(jnp.int32)

    jt_f = _flatten_idx(jt, TPAD)
    jd_f = _flatten_idx(jd, ND)
    ct_f = _flatten_idx(ct, TPAD)
    cr_f = _flatten_idx(cr, ND)

    p_jt = _pad_table(emb_job_title)
    p_jd = _pad_table(emb_job_description)
    p_ct = _pad_table(emb_candidate_title)
    p_cr = _pad_table(emb_candidate_resume)

    sc_pool = _make_sc_pool()
    pooled = sc_pool(jt_f, jd_f, ct_f, cr_f, p_jt, p_jd, p_ct, p_cr)

    out = pl.pallas_call(
        _mlp_body,
        out_shape=jax.ShapeDtypeStruct((B, 1), jnp.float32),
    )(jt, jd, ct, cr, pooled, W1,
      b1.reshape(1, 400), W2, b2.reshape(1, 100), W3, b3.reshape(1, 1))
    return out


# speedup vs baseline: 2.2894x; 1.1547x over previous
import functools

import jax
import jax.numpy as jnp
from jax import lax
from jax.experimental import pallas as pl
from jax.experimental.pallas import tpu as pltpu
from jax.experimental.pallas import tpu_sc as plsc

V = 100000
D = 300
DP = 384
B = 1024
NSL = 19
TPAD = 24
ND = 200
C0 = 104
C1 = 96
RB = 1000


def _tail_body(t_ref, o_ref):
    o_ref[...] = jnp.concatenate(
        [t_ref[:, pl.ds(0, D - 256)],
         jnp.zeros((RB, 128 - (D - 256)), jnp.float32)], axis=1)


def _tail_table(tbl):
    return pl.pallas_call(
        _tail_body,
        grid=(V // RB,),
        in_specs=[pl.BlockSpec((RB, 128), lambda i: (i, 2))],
        out_specs=pl.BlockSpec((RB, 128), lambda i: (i, 0)),
        out_shape=jax.ShapeDtypeStruct((V, 128), jnp.float32),
    )(tbl)


def _accum_rows(buf, nrows, accs):
    def body(r, a):
        return tuple(a[i] + buf[r, pl.ds(i * 16, 16)] for i in range(NSL))
    return lax.fori_loop(0, nrows, body, accs)


def _zero_accs():
    return tuple(jnp.zeros((16,), jnp.float32) for _ in range(NSL))


def _make_sc_pool():
    info = plsc.get_sparse_core_info()
    nc, ns = info.num_cores, info.num_subcores
    nw = nc * ns
    bw = B // nw
    ng = bw // 8

    mesh = plsc.VectorSubcoreMesh(core_axis_name="c", subcore_axis_name="s")

    @functools.partial(
        pl.kernel,
        mesh=mesh,
        compiler_params=pltpu.CompilerParams(use_tc_tiling_on_sc=True,
                                             needs_layout_passes=False),
        out_type=jax.ShapeDtypeStruct((4, B, DP), jnp.float32),
        scratch_types=[
            pltpu.VMEM((B // nw * TPAD,), jnp.int32),
            pltpu.VMEM((B // nw * ND,), jnp.int32),
            pltpu.VMEM((B // nw * TPAD,), jnp.int32),
            pltpu.VMEM((B // nw * ND,), jnp.int32),
            pltpu.VMEM((C0, DP), jnp.float32),
            pltpu.VMEM((C0, DP), jnp.float32),
            pltpu.VMEM((4, 8, DP), jnp.float32),
            pltpu.SemaphoreType.DMA,
            pltpu.SemaphoreType.DMA,
            pltpu.SemaphoreType.DMA,
            pltpu.SemaphoreType.DMA,
            pltpu.SemaphoreType.DMA,
            pltpu.SemaphoreType.DMA,
        ],
    )
    def sc_pool(jt_idx, jd_idx, ct_idx, cr_idx,
                t_jt, t_jd, t_ct, t_cr,
                l_jt, l_jd, l_ct, l_cr, out,
                iv_jt, iv_jd, iv_ct, iv_cr,
                buf_a, buf_b, ostage,
                sem_a, sem_b, sem_f0, sem_f1, sem_f2, sem_f3):
        wid = lax.axis_index("s") * nc + lax.axis_index("c")
        base = wid * bw

        pltpu.sync_copy(jt_idx.at[pl.ds(base * TPAD, bw * TPAD)], iv_jt)
        pltpu.sync_copy(jd_idx.at[pl.ds(base * ND, bw * ND)], iv_jd)
        pltpu.sync_copy(ct_idx.at[pl.ds(base * TPAD, bw * TPAD)], iv_ct)
        pltpu.sync_copy(cr_idx.at[pl.ds(base * ND, bw * ND)], iv_cr)

        for t in range(4):
            for jm in range(8):
                for k in range(5):
                    ostage[t, jm, pl.ds(304 + k * 16, 16)] = (
                        jnp.zeros((16,), jnp.float32))

        stages = ((0, iv_jt, t_jt, l_jt, TPAD, 0, TPAD, 20, True),
                  (1, iv_jd, t_jd, l_jd, ND, 0, C0, C0, False),
                  (1, iv_jd, t_jd, l_jd, ND, C0, C1, C1, True),
                  (2, iv_ct, t_ct, l_ct, TPAD, 0, TPAD, 20, True),
                  (3, iv_cr, t_cr, l_cr, ND, 0, C0, C0, False),
                  (3, iv_cr, t_cr, l_cr, ND, C0, C1, C1, True))
        bufs = (buf_a, buf_b)
        sems = (sem_a, sem_b)
        fsems = (sem_f0, sem_f1, sem_f2, sem_f3)

        def gather_start(s, col, b):
            _, iv, tbl, tail, stride, off, n, _, _ = stages[s]
            ixs = iv.at[pl.ds(col * stride + off, n)]
            pltpu.make_async_copy(
                tbl.at[ixs, pl.ds(0, 256)],
                bufs[b].at[pl.ds(0, n), pl.ds(0, 256)], sems[b]).start()
            pltpu.make_async_copy(
                tail.at[ixs],
                bufs[b].at[pl.ds(0, n), pl.ds(256, 128)], sems[b]).start()

        def gather_wait(s, col, b):
            _, iv, tbl, tail, stride, off, n, _, _ = stages[s]
            ixs = iv.at[pl.ds(col * stride + off, n)]
            pltpu.make_async_copy(
                tbl.at[ixs, pl.ds(0, 256)],
                bufs[b].at[pl.ds(0, n), pl.ds(0, 256)], sems[b]).wait()
            pltpu.make_async_copy(
                tail.at[ixs],
                bufs[b].at[pl.ds(0, n), pl.ds(256, 128)], sems[b]).wait()

        def flush_cp(t, g):
            return pltpu.make_async_copy(
                ostage.at[t], out.at[t, pl.ds(base + g * 8, 8)], fsems[t])

        for t in range(4):
            flush_cp(t, 0).start()
        gather_start(0, 0, 0)

        def grp_body(g, carry):
            for t in range(4):
                flush_cp(t, g).wait()

            def col_body(jm, carry2):
                j = g * 8 + jm
                accs = _zero_accs()
                for s, (t, iv, tbl, tail, stride, off, n, na, last) in (
                        enumerate(stages)):
                    nxt = (s + 1) % len(stages)
                    ncol = j if s + 1 < len(stages) else (
                        jnp.minimum(j + 1, bw - 1))
                    gather_start(nxt, ncol, (s + 1) % 2)
                    gather_wait(s, j, s % 2)
                    accs = _accum_rows(bufs[s % 2], na, accs)
                    if last:
                        for i in range(NSL):
                            ostage[t, jm, pl.ds(i * 16, 16)] = accs[i]
                        accs = _zero_accs()
                return carry2

            lax.fori_loop(0, 8, col_body, 0)
            for t in range(4):
                flush_cp(t, g).start()
            return carry

        lax.fori_loop(0, ng, grp_body, 0)

        gather_wait(0, bw - 1, 0)
        for t in range(4):
            flush_cp(t, ng - 1).wait()

    return sc_pool


def _mlp_body(jt, jd, ct, cr, pooled, w1, b1, w2, b2, w3, b3, out):
    h = jnp.broadcast_to(b1[...], (B, 400))
    zpad = jnp.zeros((DP - D, 400), jnp.float32)
    for t, idx in enumerate((jt, jd, ct, cr)):
        cnt = jnp.sum((idx[...] != 1).astype(jnp.float32), axis=0)
        x = pooled[t] / cnt[:, None]
        w1t = jnp.concatenate([w1[pl.ds(t * D, D), :], zpad], axis=0)
        h = h + jnp.dot(x, w1t, preferred_element_type=jnp.float32)
    h = jax.nn.relu(h)
    h = jax.nn.relu(jnp.dot(h, w2[...], preferred_element_type=jnp.float32)
                    + b2[...])
    h = jax.nn.relu(jnp.dot(h, w3[...], preferred_element_type=jnp.float32)
                    + b3[...])
    out[...] = jax.nn.sigmoid(h)


def _flatten_idx(idx, npad):
    n = idx.shape[0]
    cols = idx.T
    if npad > n:
        k = jnp.arange(npad - n, dtype=jnp.int32)[None, :]
        c = jnp.arange(B, dtype=jnp.int32)[:, None]
        fill = (c * 4 + k) % V
        cols = jnp.concatenate([cols, fill], axis=1)
    return cols.reshape(-1)


def kernel(job_title, job_description, candidate_title, candidate_resume,
           emb_job_title, emb_job_description, emb_candidate_title,
           emb_candidate_resume, W1, b1, W2, b2, W3, b3):
    jt = job_title.astype(jnp.int32)
    jd = job_description.astype(jnp.int32)
    ct = candidate_title.astype(jnp.int32)
    cr = candidate_resume.astype(jnp.int32)

    jt_f = _flatten_idx(jt, TPAD)
    jd_f = _flatten_idx(jd, ND)
    ct_f = _flatten_idx(ct, TPAD)
    cr_f = _flatten_idx(cr, ND)

    l_jt = _tail_table(emb_job_title)
    l_jd = _tail_table(emb_job_description)
    l_ct = _tail_table(emb_candidate_title)
    l_cr = _tail_table(emb_candidate_resume)

    sc_pool = _make_sc_pool()
    pooled = sc_pool(jt_f, jd_f, ct_f, cr_f,
                     emb_job_title, emb_job_description,
                     emb_candidate_title, emb_candidate_resume,
                     l_jt, l_jd, l_ct, l_cr)

    out = pl.pallas_call(
        _mlp_body,
        out_shape=jax.ShapeDtypeStruct((B, 1), jnp.float32),
    )(jt, jd, ct, cr, pooled, W1,
      b1.reshape(1, 400), W2, b2.reshape(1, 100), W3, b3.reshape(1, 1))
    return out


# speedup vs baseline: 2.6663x; 1.1646x over previous
import functools

import jax
import jax.numpy as jnp
from jax import lax
from jax.experimental import pallas as pl
from jax.experimental.pallas import tpu as pltpu
from jax.experimental.pallas import tpu_sc as plsc

V = 100000
D = 300
DP = 384
B = 1024
NSL = 19
TPAD = 24
ND = 200
RB = 1000


def _tail_body(t_ref, o_ref):
    o_ref[...] = jnp.concatenate(
        [t_ref[:, pl.ds(0, D - 256)],
         jnp.zeros((RB, 128 - (D - 256)), jnp.float32)], axis=1)


def _tail_table(tbl):
    return pl.pallas_call(
        _tail_body,
        grid=(V // RB,),
        in_specs=[pl.BlockSpec((RB, 128), lambda i: (i, 2))],
        out_specs=pl.BlockSpec((RB, 128), lambda i: (i, 0)),
        out_shape=jax.ShapeDtypeStruct((V, 128), jnp.float32),
    )(tbl)


def _accum_rows(buf, nrows, accs):
    def body(r, a):
        return tuple(a[i] + buf[r, pl.ds(i * 16, 16)] for i in range(NSL))
    return lax.fori_loop(0, nrows, body, accs)


def _zero_accs():
    return tuple(jnp.zeros((16,), jnp.float32) for _ in range(NSL))


def _make_sc_pool(stride, chunks):
    info = plsc.get_sparse_core_info()
    nc, ns = info.num_cores, info.num_subcores
    nw = nc * ns
    bw = B // nw
    ngrp = bw // 8
    cmax = max(c[1] for c in chunks)
    nst = len(chunks)

    mesh = plsc.VectorSubcoreMesh(core_axis_name="c", subcore_axis_name="s")

    @functools.partial(
        pl.kernel,
        mesh=mesh,
        compiler_params=pltpu.CompilerParams(use_tc_tiling_on_sc=True,
                                             needs_layout_passes=False),
        out_type=jax.ShapeDtypeStruct((B, DP), jnp.float32),
        scratch_types=[
            pltpu.VMEM((B // nw * stride,), jnp.int32),
            pltpu.VMEM((cmax, DP), jnp.float32),
            pltpu.VMEM((cmax, DP), jnp.float32),
            pltpu.VMEM((8, DP), jnp.float32),
            pltpu.SemaphoreType.DMA,
            pltpu.SemaphoreType.DMA,
            pltpu.SemaphoreType.DMA,
        ],
    )
    def sc_pool(idx, tbl, tail, out,
                iv, buf_a, buf_b, ostage, sem_a, sem_b, sem_f):
        wid = lax.axis_index("s") * nc + lax.axis_index("c")
        base = wid * bw

        pltpu.sync_copy(idx.at[pl.ds(base * stride, bw * stride)], iv)

        for jm in range(8):
            for k in range(5):
                ostage[jm, pl.ds(304 + k * 16, 16)] = (
                    jnp.zeros((16,), jnp.float32))

        bufs = (buf_a, buf_b)
        sems = (sem_a, sem_b)

        def gather_cps(s, col, b):
            off, n, _ = chunks[s]
            ixs = iv.at[pl.ds(col * stride + off, n)]
            return (
                pltpu.make_async_copy(
                    tbl.at[ixs, pl.ds(0, 256)],
                    bufs[b].at[pl.ds(0, n), pl.ds(0, 256)], sems[b]),
                pltpu.make_async_copy(
                    tail.at[ixs],
                    bufs[b].at[pl.ds(0, n), pl.ds(256, 128)], sems[b]))

        def gather_start(s, col, b):
            for cp in gather_cps(s, col, b):
                cp.start()

        def gather_wait(s, col, b):
            for cp in gather_cps(s, col, b):
                cp.wait()

        def flush_cp(g):
            return pltpu.make_async_copy(
                ostage, out.at[pl.ds(base + g * 8, 8)], sem_f)

        flush_cp(0).start()
        gather_start(0, 0, 0)

        def grp_body(g, carry):
            flush_cp(g).wait()

            unroll = 2 if nst % 2 else 1

            def col_body(ji, carry2):
                for u in range(unroll):
                    jm = ji * unroll + u
                    j = g * 8 + jm
                    accs = _zero_accs()
                    for s, (off, n, na) in enumerate(chunks):
                        p = (u * nst + s) % 2
                        nxt = (s + 1) % nst
                        last_of_iter = (u == unroll - 1) and (s == nst - 1)
                        ncol = (jnp.minimum(j + 1, bw - 1)
                                if last_of_iter else
                                (j if s + 1 < nst else j + 1))
                        gather_start(nxt, ncol, (p + 1) % 2)
                        gather_wait(s, j, p)
                        accs = _accum_rows(bufs[p], na, accs)
                    for i in range(NSL):
                        ostage[jm, pl.ds(i * 16, 16)] = accs[i]
                return carry2

            lax.fori_loop(0, 8 // unroll, col_body, 0)
            flush_cp(g).start()
            return carry

        lax.fori_loop(0, ngrp, grp_body, 0)

        gather_wait(0, bw - 1, 0)
        flush_cp(ngrp - 1).wait()

    return sc_pool


def _mlp_body(jt, jd, ct, cr, p0, p1, p2, p3,
              w1, b1, w2, b2, w3, b3, out):
    h = jnp.broadcast_to(b1[...], (B, 400))
    zpad = jnp.zeros((DP - D, 400), jnp.float32)
    for t, (idx, pooled) in enumerate(
            zip((jt, jd, ct, cr), (p0, p1, p2, p3))):
        cnt = jnp.sum((idx[...] != 1).astype(jnp.float32), axis=0)
        x = pooled[...] / cnt[:, None]
        w1t = jnp.concatenate([w1[pl.ds(t * D, D), :], zpad], axis=0)
        h = h + jnp.dot(x, w1t, preferred_element_type=jnp.float32)
    h = jax.nn.relu(h)
    h = jax.nn.relu(jnp.dot(h, w2[...], preferred_element_type=jnp.float32)
                    + b2[...])
    h = jax.nn.relu(jnp.dot(h, w3[...], preferred_element_type=jnp.float32)
                    + b3[...])
    out[...] = jax.nn.sigmoid(h)


def _flatten_idx(idx, npad):
    n = idx.shape[0]
    cols = idx.T
    if npad > n:
        k = jnp.arange(npad - n, dtype=jnp.int32)[None, :]
        c = jnp.arange(B, dtype=jnp.int32)[:, None]
        fill = (c * 4 + k) % V
        cols = jnp.concatenate([cols, fill], axis=1)
    return cols.reshape(-1)


_TITLE_CHUNKS = ((0, TPAD, 20),)
_DESC_CHUNKS = ((0, 104, 104), (104, 96, 96))


def kernel(job_title, job_description, candidate_title, candidate_resume,
           emb_job_title, emb_job_description, emb_candidate_title,
           emb_candidate_resume, W1, b1, W2, b2, W3, b3):
    jt = job_title.astype(jnp.int32)
    jd = job_description.astype(jnp.int32)
    ct = candidate_title.astype(jnp.int32)
    cr = candidate_resume.astype(jnp.int32)

    sc_title = _make_sc_pool(TPAD, _TITLE_CHUNKS)
    sc_desc = _make_sc_pool(ND, _DESC_CHUNKS)

    pooled = []
    for idx, npad, tbl, sc in (
            (jt, TPAD, emb_job_title, sc_title),
            (jd, ND, emb_job_description, sc_desc),
            (ct, TPAD, emb_candidate_title, sc_title),
            (cr, ND, emb_candidate_resume, sc_desc)):
        flat = _flatten_idx(idx, npad)
        tail = _tail_table(tbl)
        pooled.append(sc(flat, tbl, tail))

    out = pl.pallas_call(
        _mlp_body,
        out_shape=jax.ShapeDtypeStruct((B, 1), jnp.float32),
    )(jt, jd, ct, cr, *pooled, W1,
      b1.reshape(1, 400), W2, b2.reshape(1, 100), W3, b3.reshape(1, 1))
    return out
